# Initial kernel scaffold; baseline (speedup 1.0000x reference)
import jax
import jax.numpy as jnp
from jax.experimental import pallas as pl


def kernel(subgraph_embeddings, batch, W1, b1, W2, b2):
    raise NotImplementedError("write your pallas kernel here")


# speedup vs baseline: 1.6456x; 1.6456x over previous
import jax
import jax.numpy as jnp
from jax.experimental import pallas as pl
from jax.experimental.pallas import tpu as pltpu

N = 320000
D = 128
H = 64
G = 2048
TEMP = 2.0
B = 512
NEG = -1e30


def _scores(x, w1, b1, w2, b2):
    h = jax.lax.dot_general(x, w1, (((1,), (0,)), ((), ())),
                            preferred_element_type=jnp.float32)
    h = jnp.maximum(h + b1, 0.0)
    sc = jnp.sum(h * w2, axis=1, keepdims=True) + b2[:, 0:1]
    return sc / TEMP


def _k1(x_ref, bidx_ref, w1_ref, b1_ref, w2_ref, b2_ref, m_ref):
    i = pl.program_id(0)
    sc = _scores(x_ref[...], w1_ref[...], b1_ref[...], w2_ref[...], b2_ref[...])
    col = jax.lax.broadcasted_iota(jnp.int32, (B, G), 1)
    oh = bidx_ref[...] == col
    masked = jnp.where(oh, sc, NEG)
    m_loc = jnp.max(masked, axis=0, keepdims=True)

    @pl.when(i == 0)
    def _():
        m_ref[...] = jnp.full((8, G), NEG, jnp.float32)

    m_ref[...] = jnp.maximum(m_ref[...], jnp.broadcast_to(m_loc, (8, G)))


def _k2(x_ref, bidx_ref, w1_ref, b1_ref, w2_ref, b2_ref, mcol_ref,
        out_ref, acc_ref, s_ref):
    i = pl.program_id(0)
    x = x_ref[...]
    sc = _scores(x, w1_ref[...], b1_ref[...], w2_ref[...], b2_ref[...])
    col = jax.lax.broadcasted_iota(jnp.int32, (B, G), 1)
    oh = (bidx_ref[...] == col).astype(jnp.float32)
    mrow = jax.lax.dot_general(oh, mcol_ref[...], (((1,), (0,)), ((), ())),
                               preferred_element_type=jnp.float32)
    e = jnp.exp(sc - mrow)

    @pl.when(i == 0)
    def _():
        acc_ref[...] = jnp.zeros((G, D), jnp.float32)
        s_ref[...] = jnp.zeros((G, 1), jnp.float32)

    acc_ref[...] += jax.lax.dot_general(oh, e * x, (((0,), (0,)), ((), ())),
                                        preferred_element_type=jnp.float32)
    s_ref[...] += jax.lax.dot_general(oh, e, (((0,), (0,)), ((), ())),
                                      preferred_element_type=jnp.float32)

    @pl.when(i == pl.num_programs(0) - 1)
    def _():
        out_ref[...] = acc_ref[...] / (s_ref[...] + 1e-8)


def kernel(subgraph_embeddings, batch, W1, b1, W2, b2):
    x = subgraph_embeddings
    nb = N // B
    bcol = batch.astype(jnp.int32).reshape(N, 1)
    b1r = b1.reshape(1, H)
    w2r = W2.reshape(1, H)
    b2r = jnp.broadcast_to(b2.reshape(1, 1), (1, 128))

    wspecs = [
        pl.BlockSpec((D, H), lambda i: (0, 0)),
        pl.BlockSpec((1, H), lambda i: (0, 0)),
        pl.BlockSpec((1, H), lambda i: (0, 0)),
        pl.BlockSpec((1, 128), lambda i: (0, 0)),
    ]
    xspec = pl.BlockSpec((B, D), lambda i: (i, 0))
    bspec = pl.BlockSpec((B, 1), lambda i: (i, 0))

    m8 = pl.pallas_call(
        _k1,
        grid=(nb,),
        in_specs=[xspec, bspec] + wspecs,
        out_specs=pl.BlockSpec((8, G), lambda i: (0, 0)),
        out_shape=jax.ShapeDtypeStruct((8, G), jnp.float32),
    )(x, bcol, W1, b1r, w2r, b2r)

    mcol = m8[0].reshape(G, 1)

    out = pl.pallas_call(
        _k2,
        grid=(nb,),
        in_specs=[xspec, bspec] + wspecs
        + [pl.BlockSpec((G, 1), lambda i: (0, 0))],
        out_specs=pl.BlockSpec((G, D), lambda i: (0, 0)),
        out_shape=jax.ShapeDtypeStruct((G, D), jnp.float32),
        scratch_shapes=[
            pltpu.VMEM((G, D), jnp.float32),
            pltpu.VMEM((G, 1), jnp.float32),
        ],
    )(x, bcol, W1, b1r, w2r, b2r, mcol)

    return out


# speedup vs baseline: 1.6605x; 1.0090x over previous
import jax
import jax.numpy as jnp
from jax.experimental import pallas as pl
from jax.experimental.pallas import tpu as pltpu

N = 320000
D = 128
H = 64
G = 2048
TEMP = 2.0
B = 512
NEG = -1e30


def _scores(x, w1, b1, w2, b2):
    h = jax.lax.dot_general(x, w1, (((1,), (0,)), ((), ())),
                            preferred_element_type=jnp.float32)
    h = jnp.maximum(h + b1, 0.0)
    sc = jnp.sum(h * w2, axis=1, keepdims=True) + b2[:, 0:1]
    return sc / TEMP


def _k1(x_ref, bidx_ref, w1_ref, b1_ref, w2_ref, b2_ref, m_ref):
    i = pl.program_id(0)
    sc = _scores(x_ref[...], w1_ref[...], b1_ref[...], w2_ref[...], b2_ref[...])
    col = jax.lax.broadcasted_iota(jnp.int32, (B, G), 1)
    oh = bidx_ref[...] == col
    masked = jnp.where(oh, sc, NEG)
    m_loc = jnp.max(masked, axis=0, keepdims=True)

    @pl.when(i == 0)
    def _():
        m_ref[...] = jnp.full((8, G), NEG, jnp.float32)

    m_ref[...] = jnp.maximum(m_ref[...], jnp.broadcast_to(m_loc, (8, G)))


def _k2(x_ref, bidx_ref, w1_ref, b1_ref, w2_ref, b2_ref, mcol_ref,
        out_ref, acc_ref, s_ref):
    i = pl.program_id(0)
    x = x_ref[...]
    sc = _scores(x, w1_ref[...], b1_ref[...], w2_ref[...], b2_ref[...])
    col = jax.lax.broadcasted_iota(jnp.int32, (B, G), 1)
    oh = (bidx_ref[...] == col).astype(jnp.bfloat16)
    mrow = jax.lax.dot_general(oh, mcol_ref[...], (((1,), (0,)), ((), ())),
                               preferred_element_type=jnp.float32)
    e = jnp.exp(sc - mrow)

    @pl.when(i == 0)
    def _():
        acc_ref[...] = jnp.zeros((G, D), jnp.float32)
        s_ref[...] = jnp.zeros((G, 1), jnp.float32)

    exb = (e * x).astype(jnp.bfloat16)
    acc_ref[...] += jax.lax.dot_general(oh, exb, (((0,), (0,)), ((), ())),
                                        preferred_element_type=jnp.float32)
    s_ref[...] += jax.lax.dot_general(oh, e.astype(jnp.bfloat16),
                                      (((0,), (0,)), ((), ())),
                                      preferred_element_type=jnp.float32)

    @pl.when(i == pl.num_programs(0) - 1)
    def _():
        out_ref[...] = acc_ref[...] / (s_ref[...] + 1e-8)


def kernel(subgraph_embeddings, batch, W1, b1, W2, b2):
    x = subgraph_embeddings
    nb = N // B
    bcol = batch.astype(jnp.int32).reshape(N, 1)
    b1r = b1.reshape(1, H)
    w2r = W2.reshape(1, H)
    b2r = jnp.broadcast_to(b2.reshape(1, 1), (1, 128))

    wspecs = [
        pl.BlockSpec((D, H), lambda i: (0, 0)),
        pl.BlockSpec((1, H), lambda i: (0, 0)),
        pl.BlockSpec((1, H), lambda i: (0, 0)),
        pl.BlockSpec((1, 128), lambda i: (0, 0)),
    ]
    xspec = pl.BlockSpec((B, D), lambda i: (i, 0))
    bspec = pl.BlockSpec((B, 1), lambda i: (i, 0))

    m8 = pl.pallas_call(
        _k1,
        grid=(nb,),
        in_specs=[xspec, bspec] + wspecs,
        out_specs=pl.BlockSpec((8, G), lambda i: (0, 0)),
        out_shape=jax.ShapeDtypeStruct((8, G), jnp.float32),
    )(x, bcol, W1, b1r, w2r, b2r)

    mcol = m8[0].reshape(G, 1).astype(jnp.bfloat16)

    out = pl.pallas_call(
        _k2,
        grid=(nb,),
        in_specs=[xspec, bspec] + wspecs
        + [pl.BlockSpec((G, 1), lambda i: (0, 0))],
        out_specs=pl.BlockSpec((G, D), lambda i: (0, 0)),
        out_shape=jax.ShapeDtypeStruct((G, D), jnp.float32),
        scratch_shapes=[
            pltpu.VMEM((G, D), jnp.float32),
            pltpu.VMEM((G, 1), jnp.float32),
        ],
    )(x, bcol, W1, b1r, w2r, b2r, mcol)

    return out


# speedup vs baseline: 3.0980x; 1.8657x over previous
import jax
import jax.numpy as jnp
from jax.experimental import pallas as pl
from jax.experimental.pallas import tpu as pltpu

N = 320000
D = 128
H = 64
G = 2048
TEMP = 2.0
B = 512
NEG = -(2.0 ** 100)


def _scores(x, w1, b1, w2, b2):
    h = jax.lax.dot_general(x, w1, (((1,), (0,)), ((), ())),
                            preferred_element_type=jnp.float32)
    h = jnp.maximum(h + b1, 0.0)
    sc = jnp.sum(h * w2, axis=1, keepdims=True) + b2[:, 0:1]
    return sc / TEMP


def _k(x_ref, bidx_ref, w1_ref, b1_ref, w2_ref, b2_ref,
       out_ref, acc_ref, s_ref, m_ref):
    i = pl.program_id(0)
    x = x_ref[...]
    sc = _scores(x, w1_ref[...], b1_ref[...], w2_ref[...], b2_ref[...])
    col = jax.lax.broadcasted_iota(jnp.int32, (B, G), 1)
    oh = bidx_ref[...] == col
    ohb = oh.astype(jnp.bfloat16)

    @pl.when(i == 0)
    def _():
        acc_ref[...] = jnp.zeros((D, G), jnp.float32)
        s_ref[...] = jnp.zeros((8, G), jnp.float32)
        m_ref[...] = jnp.full((8, G), NEG, jnp.float32)

    m_loc = jnp.max(jnp.where(oh, sc, NEG), axis=0, keepdims=True)
    m_old = m_ref[0:1]
    m_q = jnp.maximum(m_old, m_loc).astype(jnp.bfloat16)
    m_new = m_q.astype(jnp.float32)
    m_ref[0:1] = m_new
    scale = jnp.exp(m_old - m_new)

    mrow = jnp.max(jnp.where(oh, m_new, NEG), axis=1, keepdims=True)
    e = jnp.exp(sc - mrow)
    exb = (e * x).astype(jnp.bfloat16)
    acc_ref[...] = acc_ref[...] * scale + jax.lax.dot_general(
        exb, ohb, (((0,), (0,)), ((), ())),
        preferred_element_type=jnp.float32)
    s_ref[0:1] = s_ref[0:1] * scale + jax.lax.dot_general(
        e.astype(jnp.bfloat16), ohb, (((0,), (0,)), ((), ())),
        preferred_element_type=jnp.float32)

    @pl.when(i == pl.num_programs(0) - 1)
    def _():
        out_ref[...] = acc_ref[...] / (s_ref[0:1] + 1e-8)


def kernel(subgraph_embeddings, batch, W1, b1, W2, b2):
    x = subgraph_embeddings
    nb = N // B
    bcol = batch.astype(jnp.int32).reshape(N, 1)
    b1r = b1.reshape(1, H)
    w2r = W2.reshape(1, H)
    b2r = jnp.broadcast_to(b2.reshape(1, 1), (1, 128))

    out_t = pl.pallas_call(
        _k,
        grid=(nb,),
        in_specs=[
            pl.BlockSpec((B, D), lambda i: (i, 0)),
            pl.BlockSpec((B, 1), lambda i: (i, 0)),
            pl.BlockSpec((D, H), lambda i: (0, 0)),
            pl.BlockSpec((1, H), lambda i: (0, 0)),
            pl.BlockSpec((1, H), lambda i: (0, 0)),
            pl.BlockSpec((1, 128), lambda i: (0, 0)),
        ],
        out_specs=pl.BlockSpec((D, G), lambda i: (0, 0)),
        out_shape=jax.ShapeDtypeStruct((D, G), jnp.float32),
        scratch_shapes=[
            pltpu.VMEM((D, G), jnp.float32),
            pltpu.VMEM((8, G), jnp.float32),
            pltpu.VMEM((8, G), jnp.float32),
        ],
    )(x, bcol, W1, b1r, w2r, b2r)

    return out_t.T


# speedup vs baseline: 3.5041x; 1.1311x over previous
import jax
import jax.numpy as jnp
from jax.experimental import pallas as pl
from jax.experimental.pallas import tpu as pltpu

N = 320000
D = 128
H = 64
G = 2048
TEMP = 2.0
B = 512
NEG = -(2.0 ** 100)


def _scores(x, w1, b1, w2, b2):
    h = jax.lax.dot_general(x, w1, (((1,), (0,)), ((), ())),
                            preferred_element_type=jnp.float32)
    h = jnp.maximum(h + b1, 0.0)
    sc = jnp.sum(h * w2, axis=1, keepdims=True) + b2[:, 0:1]
    return sc / TEMP


def _k(x_ref, bidx_ref, w1_ref, b1_ref, w2_ref, b2_ref,
       out_ref, acc_ref, s_ref, m_ref):
    i = pl.program_id(0)
    x = x_ref[...]
    sc = _scores(x, w1_ref[...], b1_ref[...], w2_ref[...], b2_ref[...])
    col = jax.lax.broadcasted_iota(jnp.int32, (B, G), 1)
    oh = bidx_ref[...] == col
    ohb = oh.astype(jnp.bfloat16)

    @pl.when(i == 0)
    def _():
        acc_ref[...] = jnp.zeros((D, G), jnp.float32)
        s_ref[...] = jnp.zeros((8, G), jnp.float32)
        m_ref[...] = jnp.full((8, G), NEG, jnp.float32)

    m_loc = jnp.max(jnp.where(oh, sc, NEG), axis=0, keepdims=True)
    m_old = m_ref[0:1]
    m_q = jnp.maximum(m_old, m_loc).astype(jnp.bfloat16)
    m_new = m_q.astype(jnp.float32)
    m_ref[0:1] = m_new
    scale = jnp.exp(m_old - m_new)

    mb = jnp.max(sc)
    e = jnp.exp(sc - mb)
    cscale = jnp.where(m_new == NEG, 0.0, jnp.exp(mb - m_new))
    exb = (e * x).astype(jnp.bfloat16)
    acc_ref[...] = acc_ref[...] * scale + cscale * jax.lax.dot_general(
        exb, ohb, (((0,), (0,)), ((), ())),
        preferred_element_type=jnp.float32)
    s_ref[0:1] = s_ref[0:1] * scale + cscale * jax.lax.dot_general(
        e.astype(jnp.bfloat16), ohb, (((0,), (0,)), ((), ())),
        preferred_element_type=jnp.float32)

    @pl.when(i == pl.num_programs(0) - 1)
    def _():
        out_ref[...] = acc_ref[...] / (s_ref[0:1] + 1e-8)


def kernel(subgraph_embeddings, batch, W1, b1, W2, b2):
    x = subgraph_embeddings
    nb = N // B
    bcol = batch.astype(jnp.int32).reshape(N, 1)
    b1r = b1.reshape(1, H)
    w2r = W2.reshape(1, H)
    b2r = jnp.broadcast_to(b2.reshape(1, 1), (1, 128))

    out_t = pl.pallas_call(
        _k,
        grid=(nb,),
        in_specs=[
            pl.BlockSpec((B, D), lambda i: (i, 0)),
            pl.BlockSpec((B, 1), lambda i: (i, 0)),
            pl.BlockSpec((D, H), lambda i: (0, 0)),
            pl.BlockSpec((1, H), lambda i: (0, 0)),
            pl.BlockSpec((1, H), lambda i: (0, 0)),
            pl.BlockSpec((1, 128), lambda i: (0, 0)),
        ],
        out_specs=pl.BlockSpec((D, G), lambda i: (0, 0)),
        out_shape=jax.ShapeDtypeStruct((D, G), jnp.float32),
        scratch_shapes=[
            pltpu.VMEM((D, G), jnp.float32),
            pltpu.VMEM((8, G), jnp.float32),
            pltpu.VMEM((8, G), jnp.float32),
        ],
    )(x, bcol, W1, b1r, w2r, b2r)

    return out_t.T


# speedup vs baseline: 3.6963x; 1.0548x over previous
import jax
import jax.numpy as jnp
from jax.experimental import pallas as pl
from jax.experimental.pallas import tpu as pltpu

N = 320000
D = 128
H = 64
G = 2048
TEMP = 2.0
B = 640
NEG = -(2.0 ** 100)


def _scores(x, w1, b1, w2, b2):
    h = jax.lax.dot_general(x, w1, (((1,), (0,)), ((), ())),
                            preferred_element_type=jnp.float32)
    h = jnp.maximum(h + b1, 0.0)
    sc = jnp.sum(h * w2, axis=1, keepdims=True) + b2[:, 0:1]
    return sc / TEMP


def _k(x_ref, bidx_ref, w1_ref, b1_ref, w2_ref, b2_ref,
       out_ref, acc_ref, s_ref, m_ref):
    i = pl.program_id(0)
    x = x_ref[...]
    sc = _scores(x, w1_ref[...], b1_ref[...], w2_ref[...], b2_ref[...])
    col = jax.lax.broadcasted_iota(jnp.int32, (B, G), 1)
    oh = bidx_ref[...] == col
    ohb = oh.astype(jnp.bfloat16)

    @pl.when(i == 0)
    def _():
        acc_ref[...] = jnp.zeros((D, G), jnp.float32)
        s_ref[...] = jnp.zeros((8, G), jnp.float32)
        m_ref[...] = jnp.full((8, G), NEG, jnp.float32)

    m_loc = jnp.max(jnp.where(oh, sc, NEG), axis=0, keepdims=True)
    m_old = m_ref[0:1]
    m_q = jnp.maximum(m_old, m_loc).astype(jnp.bfloat16)
    m_new = m_q.astype(jnp.float32)
    m_ref[0:1] = m_new
    scale = jnp.exp(m_old - m_new)

    mb = jnp.max(sc)
    e = jnp.exp(sc - mb)
    cscale = jnp.where(m_new == NEG, 0.0, jnp.exp(mb - m_new))
    exb = (e * x).astype(jnp.bfloat16)
    acc_ref[...] = acc_ref[...] * scale + cscale * jax.lax.dot_general(
        exb, ohb, (((0,), (0,)), ((), ())),
        preferred_element_type=jnp.float32)
    s_ref[0:1] = s_ref[0:1] * scale + cscale * jax.lax.dot_general(
        e.astype(jnp.bfloat16), ohb, (((0,), (0,)), ((), ())),
        preferred_element_type=jnp.float32)

    @pl.when(i == pl.num_programs(0) - 1)
    def _():
        out_ref[...] = acc_ref[...] / (s_ref[0:1] + 1e-8)


def kernel(subgraph_embeddings, batch, W1, b1, W2, b2):
    x = subgraph_embeddings
    nb = N // B
    bcol = batch.astype(jnp.int32).reshape(N, 1)
    b1r = b1.reshape(1, H)
    w2r = W2.reshape(1, H)
    b2r = jnp.broadcast_to(b2.reshape(1, 1), (1, 128))

    out_t = pl.pallas_call(
        _k,
        grid=(nb,),
        in_specs=[
            pl.BlockSpec((B, D), lambda i: (i, 0)),
            pl.BlockSpec((B, 1), lambda i: (i, 0)),
            pl.BlockSpec((D, H), lambda i: (0, 0)),
            pl.BlockSpec((1, H), lambda i: (0, 0)),
            pl.BlockSpec((1, H), lambda i: (0, 0)),
            pl.BlockSpec((1, 128), lambda i: (0, 0)),
        ],
        out_specs=pl.BlockSpec((D, G), lambda i: (0, 0)),
        out_shape=jax.ShapeDtypeStruct((D, G), jnp.float32),
        scratch_shapes=[
            pltpu.VMEM((D, G), jnp.float32),
            pltpu.VMEM((8, G), jnp.float32),
            pltpu.VMEM((8, G), jnp.float32),
        ],
    )(x, bcol, W1, b1r, w2r, b2r)

    return out_t.T
